# baseline (speedup 1.0000x reference)
import functools

import jax
import jax.numpy as jnp
from jax import lax
from jax.experimental import pallas as pl
from jax.experimental.pallas import tpu as pltpu
from jax.experimental.pallas import tpu_sc as plsc

_VOCAB = 1_000_000
_EMBED_DIM = 64
_IDX_COLS = 128
_ROWS_PER_STEP = 8


@functools.lru_cache(maxsize=None)
def _build(num_rows: int, embed_dim: int):
    nc, ns = 2, 16
    nw = nc * ns
    assert num_rows % (nw * _ROWS_PER_STEP) == 0
    rows_per_w = num_rows // nw
    steps = rows_per_w // _ROWS_PER_STEP
    chunk = _ROWS_PER_STEP * _IDX_COLS

    mesh = plsc.VectorSubcoreMesh(core_axis_name="c", subcore_axis_name="s")

    @functools.partial(
        pl.kernel,
        mesh=mesh,
        out_type=jax.ShapeDtypeStruct((num_rows * _IDX_COLS, embed_dim),
                                      jnp.float32),
        scratch_types=[
            pltpu.VMEM((_ROWS_PER_STEP, _IDX_COLS), jnp.int32),
            pltpu.VMEM((chunk, embed_dim), jnp.float32),
            pltpu.SemaphoreType.DMA,
        ],
        compiler_params=pltpu.CompilerParams(use_tc_tiling_on_sc=False),
    )
    def gather_kernel(idx_hbm, table_hbm, out_hbm, idx_v, rows_v, sem):
        wid = lax.axis_index("s") * nc + lax.axis_index("c")
        row0 = wid * rows_per_w

        def body(i, _):
            r = row0 + i * _ROWS_PER_STEP
            pltpu.sync_copy(idx_hbm.at[pl.ds(r, _ROWS_PER_STEP)], idx_v)
            copies = []
            for j in range(_ROWS_PER_STEP):
                copies.append(pltpu.async_copy(
                    table_hbm.at[idx_v.at[j]],
                    rows_v.at[pl.ds(j * _IDX_COLS, _IDX_COLS)],
                    sem))
            for c in copies:
                c.wait()
            pltpu.sync_copy(rows_v, out_hbm.at[pl.ds(r * _IDX_COLS, chunk)])
            return 0

        lax.fori_loop(0, steps, body, 0)

    return gather_kernel


def kernel(x, table):
    b, h = x.shape
    n = b * h
    idx2d = x.reshape(n // _IDX_COLS, _IDX_COLS)
    out = _build(n // _IDX_COLS, table.shape[1])(idx2d, table)
    return out.reshape(b, h, table.shape[1])


# speedup vs baseline: 1.0143x; 1.0143x over previous
import functools

import jax
import jax.numpy as jnp
from jax import lax
from jax.experimental import pallas as pl
from jax.experimental.pallas import tpu as pltpu
from jax.experimental.pallas import tpu_sc as plsc

_IDX_COLS = 128
_NBUF = 10
_LAG = 5


@functools.lru_cache(maxsize=None)
def _build(num_rows: int, embed_dim: int):
    nc, ns = 2, 16
    nw = nc * ns
    assert num_rows % (nw * _NBUF) == 0
    rows_per_w = num_rows // nw
    outer = rows_per_w // _NBUF

    mesh = plsc.VectorSubcoreMesh(core_axis_name="c", subcore_axis_name="s")

    scratch = (
        [pltpu.VMEM((rows_per_w, _IDX_COLS), jnp.int32)]
        + [pltpu.VMEM((_IDX_COLS, embed_dim), jnp.float32)
           for _ in range(_NBUF)]
        + [pltpu.SemaphoreType.DMA for _ in range(2 * _NBUF)]
    )

    @functools.partial(
        pl.kernel,
        mesh=mesh,
        out_type=jax.ShapeDtypeStruct((num_rows * _IDX_COLS, embed_dim),
                                      jnp.float32),
        scratch_types=scratch,
        compiler_params=pltpu.CompilerParams(use_tc_tiling_on_sc=False),
    )
    def gather_kernel(idx_hbm, table_hbm, out_hbm, idx_v, *bufs_and_sems):
        rows = bufs_and_sems[:_NBUF]
        gsem = bufs_and_sems[_NBUF:2 * _NBUF]
        osem = bufs_and_sems[2 * _NBUF:]

        wid = lax.axis_index("s") * nc + lax.axis_index("c")
        row0 = wid * rows_per_w

        pltpu.sync_copy(idx_hbm.at[pl.ds(row0, rows_per_w)], idx_v)

        def fire_gather(t, b):
            pltpu.make_async_copy(
                table_hbm.at[idx_v.at[t]], rows[b], gsem[b]).start()

        def wait_gather(b):
            pltpu.make_async_copy(
                table_hbm.at[idx_v.at[0]], rows[b], gsem[b]).wait()

        def fire_out(t, b):
            pltpu.make_async_copy(
                rows[b],
                out_hbm.at[pl.ds((row0 + t) * _IDX_COLS, _IDX_COLS)],
                osem[b]).start()

        def wait_out(b):
            pltpu.make_async_copy(
                rows[b], out_hbm.at[pl.ds(0, _IDX_COLS)], osem[b]).wait()

        for b in range(_NBUF):
            fire_gather(b, b)
            if b >= _LAG:
                t2 = b - _LAG
                wait_gather(t2)
                fire_out(t2, t2)

        def body(g, _):
            t0 = g * _NBUF
            for b in range(_NBUF):
                t = t0 + b
                wait_out(b)
                fire_gather(t, b)
                b2 = (b - _LAG) % _NBUF
                wait_gather(b2)
                fire_out(t - _LAG, b2)
            return 0

        lax.fori_loop(1, outer, body, 0)

        tail0 = rows_per_w - _LAG
        for i in range(_LAG):
            b = (tail0 + i) % _NBUF
            wait_gather(b)
            fire_out(tail0 + i, b)
        for b in range(_NBUF):
            wait_out(b)

    return gather_kernel


def kernel(x, table):
    b, h = x.shape
    n = b * h
    idx2d = x.reshape(n // _IDX_COLS, _IDX_COLS)
    out = _build(n // _IDX_COLS, table.shape[1])(idx2d, table)
    return out.reshape(b, h, table.shape[1])


# speedup vs baseline: 1.0145x; 1.0002x over previous
import functools

import jax
import jax.numpy as jnp
from jax import lax
from jax.experimental import pallas as pl
from jax.experimental.pallas import tpu as pltpu
from jax.experimental.pallas import tpu_sc as plsc

_BBLK = 128
_NBUF = 10
_LAG = 5


@functools.lru_cache(maxsize=None)
def _build(batch: int, hist: int, embed_dim: int):
    nc, ns = 2, 16
    nw = nc * ns
    assert batch == nw * _BBLK and hist % _NBUF == 0

    mesh = plsc.VectorSubcoreMesh(core_axis_name="c", subcore_axis_name="s")

    scratch = (
        [pltpu.VMEM((hist, _BBLK), jnp.int32)]
        + [pltpu.VMEM((_BBLK, embed_dim), jnp.float32)
           for _ in range(_NBUF)]
        + [pltpu.SemaphoreType.DMA for _ in range(2 * _NBUF)]
    )

    @functools.partial(
        pl.kernel,
        mesh=mesh,
        out_type=jax.ShapeDtypeStruct((batch, hist, embed_dim), jnp.float32),
        scratch_types=scratch,
        compiler_params=pltpu.CompilerParams(use_tc_tiling_on_sc=False),
    )
    def gather_kernel(xt_hbm, table_hbm, out_hbm, idx_v, *bufs_and_sems):
        rows = bufs_and_sems[:_NBUF]
        gsem = bufs_and_sems[_NBUF:2 * _NBUF]
        osem = bufs_and_sems[2 * _NBUF:]

        wid = lax.axis_index("s") * nc + lax.axis_index("c")
        b0 = wid * _BBLK

        pltpu.sync_copy(xt_hbm.at[:, pl.ds(b0, _BBLK)], idx_v)

        def fire_gather(t, b):
            pltpu.make_async_copy(
                table_hbm.at[idx_v.at[t]], rows[b], gsem[b]).start()

        def wait_gather(b):
            pltpu.make_async_copy(
                table_hbm.at[idx_v.at[0]], rows[b], gsem[b]).wait()

        def fire_out(t, b):
            pltpu.make_async_copy(
                rows[b], out_hbm.at[pl.ds(b0, _BBLK), t], osem[b]).start()

        def wait_out(b):
            pltpu.make_async_copy(
                rows[b], out_hbm.at[pl.ds(b0, _BBLK), 0], osem[b]).wait()

        for b in range(_NBUF):
            fire_gather(b, b)
            if b >= _LAG:
                t2 = b - _LAG
                wait_gather(t2)
                fire_out(t2, t2)

        def body(g, _):
            t0 = g * _NBUF
            for b in range(_NBUF):
                t = t0 + b
                wait_out(b)
                fire_gather(t, b)
                b2 = (b - _LAG) % _NBUF
                wait_gather(b2)
                fire_out(t - _LAG, b2)
            return 0

        lax.fori_loop(1, hist // _NBUF, body, 0)

        tail0 = hist - _LAG
        for i in range(_LAG):
            b = (tail0 + i) % _NBUF
            wait_gather(b)
            fire_out(tail0 + i, b)
        for b in range(_NBUF):
            wait_out(b)

    return gather_kernel


def kernel(x, table):
    b, h = x.shape
    xt = jnp.transpose(x)
    return _build(b, h, table.shape[1])(xt, table)
